# baseline (speedup 1.0000x reference)
import jax
import jax.numpy as jnp
from jax.experimental import pallas as pl

N_NODES = 10000
NP = 10240
E_RAW = 160000
E2 = E_RAW + N_NODES
BE = 2048
E2P = ((E2 + BE - 1) // BE) * BE


def _mm_body(x_ref, wl_ref, bl_ref, wr_ref, br_ref, xl_ref, xr_ref):
    xb = x_ref[...]
    xl_ref[...] = jnp.dot(xb, wl_ref[...], preferred_element_type=jnp.float32) + bl_ref[...]
    xr_ref[...] = jnp.dot(xb, wr_ref[...], preferred_element_type=jnp.float32) + br_ref[...]


def _matmul2(x, wl, bl, wr, br, bn=256):
    np_, din = x.shape
    c = wl.shape[1]
    return pl.pallas_call(
        _mm_body,
        grid=(np_ // bn,),
        in_specs=[
            pl.BlockSpec((bn, din), lambda i: (i, 0)),
            pl.BlockSpec((din, c), lambda i: (0, 0)),
            pl.BlockSpec((1, c), lambda i: (0, 0)),
            pl.BlockSpec((din, c), lambda i: (0, 0)),
            pl.BlockSpec((1, c), lambda i: (0, 0)),
        ],
        out_specs=[
            pl.BlockSpec((bn, c), lambda i: (i, 0)),
            pl.BlockSpec((bn, c), lambda i: (i, 0)),
        ],
        out_shape=[jax.ShapeDtypeStruct((np_, c), jnp.float32)] * 2,
    )(x, wl, bl.reshape(1, -1), wr, br.reshape(1, -1))


def _alpha_body(g_ref, r_ref, att_ref, out_ref):
    z = g_ref[...] + r_ref[...]
    e = jnp.maximum(z, 0.2 * z)
    out_ref[...] = jnp.dot(e, att_ref[...], preferred_element_type=jnp.float32)


def _alpha(g, r, attv):
    c = g.shape[1]
    att8 = jnp.zeros((c, 8), jnp.float32).at[:, 0].set(attv)
    out = pl.pallas_call(
        _alpha_body,
        grid=(E2P // BE,),
        in_specs=[
            pl.BlockSpec((BE, c), lambda i: (i, 0)),
            pl.BlockSpec((BE, c), lambda i: (i, 0)),
            pl.BlockSpec((c, 8), lambda i: (0, 0)),
        ],
        out_specs=pl.BlockSpec((BE, 8), lambda i: (i, 0)),
        out_shape=jax.ShapeDtypeStruct((E2P, 8), jnp.float32),
    )(g, r, att8)
    return out[:, 0]


def _scale_body(g_ref, w_ref, out_ref):
    out_ref[...] = g_ref[...] * w_ref[...]


def _scale(g, w):
    c = g.shape[1]
    return pl.pallas_call(
        _scale_body,
        grid=(E2P // BE,),
        in_specs=[
            pl.BlockSpec((BE, c), lambda i: (i, 0)),
            pl.BlockSpec((BE, 1), lambda i: (i, 0)),
        ],
        out_specs=pl.BlockSpec((BE, c), lambda i: (i, 0)),
        out_shape=jax.ShapeDtypeStruct((E2P, c), jnp.float32),
    )(g, w.reshape(-1, 1))


def _epi_body_relu(a_ref, b_ref, o_ref):
    o_ref[...] = jnp.maximum(a_ref[...] + b_ref[...], 0.0)


def _epi_body_sig(a_ref, b_ref, o_ref):
    z = a_ref[...] + b_ref[...]
    o_ref[...] = 1.0 / (1.0 + jnp.exp(-z))


def _epilogue(acc, bias, act, bn=256):
    c = acc.shape[1]
    body = _epi_body_relu if act == "relu" else _epi_body_sig
    return pl.pallas_call(
        body,
        grid=(NP // bn,),
        in_specs=[
            pl.BlockSpec((bn, c), lambda i: (i, 0)),
            pl.BlockSpec((1, c), lambda i: (0, 0)),
        ],
        out_specs=pl.BlockSpec((bn, c), lambda i: (i, 0)),
        out_shape=jax.ShapeDtypeStruct((NP, c), jnp.float32),
    )(acc, bias.reshape(1, -1))


_LAYER_C = [128, 512, 1024, 512, 256, 128, 16]


def kernel(x, edge_index, params):
    loop = jnp.arange(N_NODES, dtype=edge_index.dtype)
    src = jnp.concatenate([edge_index[0], loop])
    dst = jnp.concatenate([edge_index[1], loop])
    padix = jnp.full((E2P - E2,), N_NODES, jnp.int32)
    srcp = jnp.concatenate([src.astype(jnp.int32), padix])
    dstp = jnp.concatenate([dst.astype(jnp.int32), padix])

    h = jnp.pad(x, ((0, NP - N_NODES), (0, 4)))
    nl = len(_LAYER_C)
    for i, c in enumerate(_LAYER_C):
        wl, bl, wr, br, att, bias = params[i]
        attv = att.reshape(-1)
        if i == 0:
            wl = jnp.pad(wl, ((0, 4), (0, 0)))
            wr = jnp.pad(wr, ((0, 4), (0, 0)))
        if i == nl - 1:
            wl = jnp.pad(wl, ((0, 0), (0, 15)))
            wr = jnp.pad(wr, ((0, 0), (0, 15)))
            bl = jnp.pad(bl, (0, 15))
            br = jnp.pad(br, (0, 15))
            attv = jnp.pad(attv, (0, 15))
            bias = jnp.pad(bias, (0, 15))
        xl, xr = _matmul2(h, wl, bl, wr, br)
        g = xl[srcp]
        r = xr[dstp]
        alpha = _alpha(g, r, attv)
        amax = jax.ops.segment_max(alpha, dstp, num_segments=NP)
        amax = jnp.where(jnp.isneginf(amax), 0.0, amax)
        ex = jnp.exp(alpha - amax[dstp])
        den = jax.ops.segment_sum(ex, dstp, num_segments=NP)
        w = ex / (den[dstp] + 1e-16)
        gw = _scale(g, w)
        acc = jax.ops.segment_sum(gw, dstp, num_segments=NP)
        act = "relu" if i < nl - 1 else "sigmoid"
        h = _epilogue(acc, bias, act)
    return h[:N_NODES, :1]
